# baseline (speedup 1.0000x reference)
import functools

import jax
import jax.numpy as jnp
from jax.experimental import pallas as pl
from jax.experimental.pallas import tpu as pltpu


def _round_up(x, m):
    return ((x + m - 1) // m) * m


_BF16_ONE_LO = 0x00003F80
_BF16_ONE_HI = 0x3F800000


def _make_body(f, d_max, d_out):
    def _body(idx_ref, tab_ref, out_ref):
        hn = idx_ref.shape[0]
        col = jax.lax.broadcasted_iota(jnp.int32, (hn, d_max), 1)
        for g in range(f):
            even = jnp.where(col == idx_ref[:, g:g + 1], _BF16_ONE_LO, 0)
            odd = jnp.where(col == idx_ref[:, f + g:f + g + 1], _BF16_ONE_HI, 0)
            oh = pltpu.bitcast(even | odd, jnp.bfloat16)
            out_ref[:, g * d_out:(g + 1) * d_out] = jnp.dot(
                oh, tab_ref[g * d_max:(g + 1) * d_max, :],
                preferred_element_type=jnp.float32)
    return _body


@functools.partial(jax.jit, static_argnames=("row_tile",))
def kernel(indices, tables, *, row_tile=1024):
    n, f = indices.shape
    f_tab, d_max, d_out = tables.shape
    assert f_tab == f

    tn = min(_round_up(n, 16), _round_up(int(row_tile), 16))
    num_n = pl.cdiv(n, tn)
    n_pad = num_n * tn

    idx = indices.astype(jnp.int32)
    if n_pad != n:
        idx = jnp.pad(idx, ((0, n_pad - n), (0, 0)))
    idx = idx.reshape(n_pad // 2, 2 * f)
    tab = tables.astype(jnp.bfloat16).reshape(f * d_max, d_out)

    return pl.pallas_call(
        _make_body(f, d_max, d_out),
        grid=(num_n,),
        in_specs=[
            pl.BlockSpec((tn // 2, 2 * f), lambda ni: (ni, 0)),
            pl.BlockSpec((f * d_max, d_out), lambda ni: (0, 0)),
        ],
        out_shape=jax.ShapeDtypeStruct((n, f * d_out), tables.dtype),
        out_specs=pl.BlockSpec((tn, f * d_out), lambda ni: (ni, 0)),
        compiler_params=pltpu.CompilerParams(
            dimension_semantics=("parallel",)),
    )(idx, tab)


# speedup vs baseline: 1.3352x; 1.3352x over previous
import functools

import jax
import jax.numpy as jnp
from jax.experimental import pallas as pl
from jax.experimental.pallas import tpu as pltpu


def _round_up(x, m):
    return ((x + m - 1) // m) * m


def _make_body(f, d_max, d_out):
    def _body(idx_ref, tab_ref, out_ref):
        tn = idx_ref.shape[0]
        col = jax.lax.broadcasted_iota(jnp.int32, (tn, d_max), 1)
        for g in range(f):
            oh = (col == idx_ref[:, g:g + 1]).astype(jnp.bfloat16)
            out_ref[:, g * d_out:(g + 1) * d_out] = jnp.dot(
                oh, tab_ref[g * d_max:(g + 1) * d_max, :],
                preferred_element_type=jnp.float32)
    return _body


@functools.partial(jax.jit, static_argnames=("row_tile",))
def kernel(indices, tables, *, row_tile=2048):
    n, f = indices.shape
    f_tab, d_max, d_out = tables.shape
    assert f_tab == f

    tn = min(_round_up(n, 8), _round_up(int(row_tile), 8))
    num_n = pl.cdiv(n, tn)
    n_pad = num_n * tn

    idx = indices.astype(jnp.int32)
    if n_pad != n:
        idx = jnp.pad(idx, ((0, n_pad - n), (0, 0)))
    tab = tables.astype(jnp.bfloat16).reshape(f * d_max, d_out)

    return pl.pallas_call(
        _make_body(f, d_max, d_out),
        grid=(num_n,),
        in_specs=[
            pl.BlockSpec((tn, f), lambda ni: (ni, 0)),
            pl.BlockSpec((f * d_max, d_out), lambda ni: (0, 0)),
        ],
        out_shape=jax.ShapeDtypeStruct((n, f * d_out), tables.dtype),
        out_specs=pl.BlockSpec((tn, f * d_out), lambda ni: (ni, 0)),
        compiler_params=pltpu.CompilerParams(
            dimension_semantics=("parallel",)),
    )(idx, tab)


# speedup vs baseline: 1.4974x; 1.1215x over previous
import functools

import jax
import jax.numpy as jnp
from jax.experimental import pallas as pl
from jax.experimental.pallas import tpu as pltpu


def _round_up(x, m):
    return ((x + m - 1) // m) * m


def _make_body(f, d_max, d_out):
    def _body(idx_ref, tab_ref, out_ref):
        tn = idx_ref.shape[0]
        col = jax.lax.broadcasted_iota(jnp.int32, (tn, d_max), 1)
        for g in range(f):
            oh = (col == idx_ref[:, g:g + 1]).astype(jnp.bfloat16)
            out_ref[:, g * d_out:(g + 1) * d_out] = jnp.dot(
                oh, tab_ref[g * d_max:(g + 1) * d_max, :],
                preferred_element_type=jnp.float32)
    return _body


@functools.partial(jax.jit, static_argnames=("row_tile",))
def kernel(indices, tables, *, row_tile=4096):
    n, f = indices.shape
    f_tab, d_max, d_out = tables.shape
    assert f_tab == f

    tn = min(_round_up(n, 8), _round_up(int(row_tile), 8))
    num_n = pl.cdiv(n, tn)
    n_pad = num_n * tn

    idx = indices.astype(jnp.int32)
    if n_pad != n:
        idx = jnp.pad(idx, ((0, n_pad - n), (0, 0)))
    tab = tables.astype(jnp.bfloat16).reshape(f * d_max, d_out)

    return pl.pallas_call(
        _make_body(f, d_max, d_out),
        grid=(num_n,),
        in_specs=[
            pl.BlockSpec((tn, f), lambda ni: (ni, 0)),
            pl.BlockSpec((f * d_max, d_out), lambda ni: (0, 0)),
        ],
        out_shape=jax.ShapeDtypeStruct((n, f * d_out), tables.dtype),
        out_specs=pl.BlockSpec((tn, f * d_out), lambda ni: (ni, 0)),
        compiler_params=pltpu.CompilerParams(
            dimension_semantics=("parallel",)),
    )(idx, tab)


# speedup vs baseline: 1.5306x; 1.0222x over previous
import functools

import jax
import jax.numpy as jnp
from jax.experimental import pallas as pl
from jax.experimental.pallas import tpu as pltpu


def _round_up(x, m):
    return ((x + m - 1) // m) * m


def _make_body(f, d_max, d_out):
    def _body(idx_ref, tab_ref, out_ref):
        tn = idx_ref.shape[0]
        col = jax.lax.broadcasted_iota(jnp.int32, (tn, d_max), 1)
        for g in range(f):
            oh = (col == idx_ref[:, g:g + 1]).astype(jnp.bfloat16)
            out_ref[:, g * d_out:(g + 1) * d_out] = jnp.dot(
                oh, tab_ref[g * d_max:(g + 1) * d_max, :],
                preferred_element_type=jnp.float32)
    return _body


@functools.partial(jax.jit, static_argnames=("row_tile",))
def kernel(indices, tables, *, row_tile=8192):
    n, f = indices.shape
    f_tab, d_max, d_out = tables.shape
    assert f_tab == f

    tn = min(_round_up(n, 8), _round_up(int(row_tile), 8))
    num_n = pl.cdiv(n, tn)
    n_pad = num_n * tn

    idx = indices.astype(jnp.int32)
    if n_pad != n:
        idx = jnp.pad(idx, ((0, n_pad - n), (0, 0)))
    tab = tables.astype(jnp.bfloat16).reshape(f * d_max, d_out)

    return pl.pallas_call(
        _make_body(f, d_max, d_out),
        grid=(num_n,),
        in_specs=[
            pl.BlockSpec((tn, f), lambda ni: (ni, 0)),
            pl.BlockSpec((f * d_max, d_out), lambda ni: (0, 0)),
        ],
        out_shape=jax.ShapeDtypeStruct((n, f * d_out), tables.dtype),
        out_specs=pl.BlockSpec((tn, f * d_out), lambda ni: (ni, 0)),
        compiler_params=pltpu.CompilerParams(
            dimension_semantics=("parallel",)),
    )(idx, tab)
